# baseline (speedup 1.0000x reference)
import functools

import jax
import jax.numpy as jnp
from jax import lax
from jax.experimental import pallas as pl
from jax.experimental.pallas import tpu as pltpu
from jax.experimental.pallas import tpu_sc as plsc

N_NODES = 100000
D = 128
S1 = 10
S2 = 10
H = 128
ATT = 64
E1 = 128
E2 = 128
B = 1024
L = 50
NC_OUT = 50

NW = 32
BPW = B // NW
N1 = BPW * S2
N2 = N1 * S1
LPW = B * L // NW
RCH = 128


def _sc_gather(nodes, neigh_flat, seq_flat, embed_table, seq_embed_table):
  mesh = plsc.VectorSubcoreMesh(core_axis_name="c", subcore_axis_name="s")

  @functools.partial(
      pl.kernel,
      out_type=(
          jax.ShapeDtypeStruct((B * S2 * S1, D), jnp.float32),
          jax.ShapeDtypeStruct((B * L, D), jnp.float32),
      ),
      mesh=mesh,
      compiler_params=pltpu.CompilerParams(needs_layout_passes=False),
      scratch_types=[
          pltpu.VMEM((BPW,), jnp.int32),
          pltpu.VMEM((N1,), jnp.int32),
          pltpu.VMEM((N1,), jnp.int32),
          pltpu.VMEM((N2,), jnp.int32),
          pltpu.VMEM((N2,), jnp.int32),
          pltpu.VMEM((LPW,), jnp.int32),
          pltpu.VMEM((RCH, D), jnp.float32),
          pltpu.VMEM((RCH, D), jnp.float32),
          pltpu.SemaphoreType.DMA,
          pltpu.SemaphoreType.DMA,
      ],
  )
  def k(nodes_h, nf_h, seq_h, emb_h, semb_h, feats_h, xseq_h,
        nodes_v, idx1_v, ids1_v, idx2_v, ids2_v, seq_v, bufa, bufb,
        sema, semb):
    wid = lax.axis_index("s") * 2 + lax.axis_index("c")
    pltpu.sync_copy(nodes_h.at[pl.ds(wid * BPW, BPW)], nodes_v)
    pltpu.sync_copy(seq_h.at[pl.ds(wid * LPW, LPW)], seq_v)

    def expand_by_10(i, src_ref, dst_ref):
      j = lax.iota(jnp.int32, 16) + i * 16
      m = j // 10
      r = j - m * 10
      vals = plsc.load_gather(src_ref, [m])
      dst_ref[pl.ds(pl.multiple_of(i * 16, 16), 16)] = vals * 10 + r

    lax.fori_loop(0, N1 // 16, lambda i, c: (expand_by_10(i, nodes_v, idx1_v), c)[1], 0)
    for off, n in ((0, 128), (128, 128), (256, 64)):
      pltpu.async_copy(nf_h.at[idx1_v.at[pl.ds(off, n)]],
                       ids1_v.at[pl.ds(off, n)], sema).wait()
    lax.fori_loop(0, N2 // 16, lambda i, c: (expand_by_10(i, ids1_v, idx2_v), c)[1], 0)
    for c in range(N2 // RCH):
      pltpu.async_copy(nf_h.at[idx2_v.at[pl.ds(c * RCH, RCH)]],
                       ids2_v.at[pl.ds(c * RCH, RCH)], sema).wait()

    NFC = N2 // RCH
    NSC = LPW // RCH
    jobs = []
    for c in range(NFC):
      jobs.append((emb_h, ids2_v, c * RCH, RCH, feats_h, wid * N2 + c * RCH))
    for c in range(NSC):
      jobs.append((semb_h, seq_v, c * RCH, RCH, xseq_h, wid * LPW + c * RCH))
    jobs.append((semb_h, seq_v, NSC * RCH, LPW - NSC * RCH, xseq_h,
                 wid * LPW + NSC * RCH))

    bufs = (bufa, bufb)
    sems = (sema, semb)

    def start(j, slot):
      tab, idx, ioff, n, _, _ = jobs[j]
      return pltpu.async_copy(tab.at[idx.at[pl.ds(ioff, n)]],
                              bufs[slot].at[pl.ds(0, n)], sems[slot])

    cp = start(0, 0)
    for j in range(len(jobs)):
      nxt = start(j + 1, (j + 1) % 2) if j + 1 < len(jobs) else None
      cp.wait()
      _, _, _, n, out_h, ooff = jobs[j]
      pltpu.sync_copy(bufs[j % 2].at[pl.ds(0, n)], out_h.at[pl.ds(ooff, n)])
      cp = nxt

  return k(nodes, neigh_flat, seq_flat, embed_table, seq_embed_table)


CH = 128
GRID = B // CH
CHS = 256
GRIDS = B // CHS


def _att_pool(x3, w, b, c):
  a = jnp.tanh(
      lax.dot_general(x3, w, (((2,), (0,)), ((), ())),
                      preferred_element_type=jnp.float32) + b[None, :, :])
  sc = jnp.sum(a * c[None, :, :], axis=2)
  m = jnp.max(sc, axis=1, keepdims=True)
  e = jnp.exp(sc - m)
  p = e * (1.0 / jnp.sum(e, axis=1, keepdims=True))
  return jnp.sum(x3 * p[:, :, None], axis=1)


def _gru_cell(gi, gh, h):
  ir, iz, inn = gi[:, :H], gi[:, H:2 * H], gi[:, 2 * H:]
  hr, hz, hn = gh[:, :H], gh[:, H:2 * H], gh[:, 2 * H:]
  r = jax.nn.sigmoid(ir + hr)
  z = jax.nn.sigmoid(iz + hz)
  n = jnp.tanh(inn + r * hn)
  return (1.0 - z) * n + z * h


def _mm(x, w):
  return lax.dot_general(x, w, (((1,), (0,)), ((), ())),
                         preferred_element_type=jnp.float32)


def _graph_body(feats_ref, a1w, a1b, a1c, w1, a2w, a2b, a2c, w2, out_ref):
  f3 = feats_ref[...].reshape(CH * S2, S1, D)
  ws1 = _att_pool(f3, a1w[...], a1b[...], a1c[...])
  agg1 = jax.nn.relu(ws1 * (1.0 / S1))
  e1 = jax.nn.relu(
      lax.dot_general(agg1, w1[...], (((1,), (1,)), ((), ())),
                      preferred_element_type=jnp.float32))
  e13 = e1.reshape(CH, S2, E1)
  ws2 = _att_pool(e13, a2w[...], a2b[...], a2c[...])
  agg2 = jax.nn.relu(ws2 * (1.0 / S2))
  out_ref[...] = jax.nn.relu(
      lax.dot_general(agg2, w2[...], (((1,), (1,)), ((), ())),
                      preferred_element_type=jnp.float32))


def _seq_body(xseq_ref, graph_ref,
              wih0ft, wih0bt, bih0f, bih0b, bdhh0, bhh0,
              wih1ft, wih1bt, bih1f, bih1b, bdhh1, bhh1,
              afw, afb, afc, wgt1, wgt2, out_ref, xtm, o0f, o0b, gi):
  graph = graph_ref[...]
  x3 = xseq_ref[...].reshape(CHS, L, D)
  for t in range(L):
    xtm[t] = x3[:, t, :]
  TS = 10
  for tb in range(L // TS):
    xs = xtm[tb * TS:(tb + 1) * TS].reshape(TS * CHS, D)
    gi[pl.ds(tb * TS * CHS, TS * CHS), :3 * H] = (_mm(xs, wih0ft[...]) + bih0f[...]).astype(jnp.bfloat16)
    gi[pl.ds(tb * TS * CHS, TS * CHS), 3 * H:] = (_mm(xs, wih0bt[...]) + bih0b[...]).astype(jnp.bfloat16)

  def gru_loop(bd, bh, store):
    def step(t, carry):
      hf, hb = carry
      hcat = jnp.concatenate([hf, hb], axis=1).astype(jnp.bfloat16)
      gh = _mm(hcat, bd) + bh
      gif = gi[pl.ds(t * CHS, CHS), :3 * H].astype(jnp.float32)
      gib = gi[pl.ds((L - 1 - t) * CHS, CHS), 3 * H:].astype(jnp.float32)
      hf = _gru_cell(gif, gh[:, :3 * H], hf)
      hb = _gru_cell(gib, gh[:, 3 * H:], hb)
      if store:
        o0f[t] = hf
        o0b[L - 1 - t] = hb
      return hf, hb

    z2 = jnp.zeros((CHS, H), jnp.float32)
    return lax.fori_loop(0, L, step, (z2, z2))

  gru_loop(bdhh0[...], bhh0[...], True)

  for tb in range(L // TS):
    o01 = jnp.concatenate(
        [o0f[tb * TS:(tb + 1) * TS], o0b[tb * TS:(tb + 1) * TS]],
        axis=2).reshape(TS * CHS, 2 * H)
    gi[pl.ds(tb * TS * CHS, TS * CHS), :3 * H] = (_mm(o01, wih1ft[...]) + bih1f[...]).astype(jnp.bfloat16)
    gi[pl.ds(tb * TS * CHS, TS * CHS), 3 * H:] = (_mm(o01, wih1bt[...]) + bih1b[...]).astype(jnp.bfloat16)
  h1f, h1b = gru_loop(bdhh1[...], bhh1[...], False)
  rnn = jnp.concatenate([h1f, h1b], axis=1)
  rnn_proj = lax.dot_general(rnn, wgt1[...], (((1,), (0,)), ((), ())),
                             preferred_element_type=jnp.float32)

  def att_s(v):
    a = jnp.tanh(_mm(v, afw[...]) + afb[...])
    return _mm(a, afc[...])

  sr = att_s(rnn_proj)
  sg = att_s(graph)
  m = jnp.maximum(sr, sg)
  er = jnp.exp(sr - m)
  eg = jnp.exp(sg - m)
  ws = (er * rnn_proj + eg * graph) / (er + eg)
  out_ref[...] = lax.dot_general(ws, wgt2[...], (((1,), (0,)), ((), ())),
                                 preferred_element_type=jnp.float32)


def _full(a):
  return pl.BlockSpec(a.shape, lambda c: (0,) * a.ndim)


def _graph_call(feats, gweights):
  return pl.pallas_call(
      _graph_body,
      grid=(GRID,),
      in_specs=[pl.BlockSpec((CH * S2 * S1, D), lambda c: (c, 0))]
      + [_full(w) for w in gweights],
      out_specs=pl.BlockSpec((CH, E2), lambda c: (c, 0)),
      out_shape=jax.ShapeDtypeStruct((B, E2), jnp.float32),
      compiler_params=pltpu.CompilerParams(
          dimension_semantics=("arbitrary",),
          vmem_limit_bytes=60000 * 1024,
      ),
  )(feats, *gweights)


def _seq_call(xseq, graph, sweights):
  return pl.pallas_call(
      _seq_body,
      grid=(GRIDS,),
      in_specs=[
          pl.BlockSpec((CHS * L, D), lambda c: (c, 0)),
          pl.BlockSpec((CHS, E2), lambda c: (c, 0)),
      ] + [_full(w) for w in sweights],
      out_specs=pl.BlockSpec((CHS, NC_OUT), lambda c: (c, 0)),
      out_shape=jax.ShapeDtypeStruct((B, NC_OUT), jnp.float32),
      scratch_shapes=[
          pltpu.VMEM((L, CHS, D), jnp.float32),
          pltpu.VMEM((L, CHS, H), jnp.float32),
          pltpu.VMEM((L, CHS, H), jnp.float32),
          pltpu.VMEM((L * CHS, 6 * H), jnp.bfloat16),
      ],
      compiler_params=pltpu.CompilerParams(
          dimension_semantics=("arbitrary",),
          vmem_limit_bytes=60000 * 1024,
      ),
  )(xseq, graph, *sweights)


def kernel(nodes, seq_input, neigh_table, embed_table, seq_embed_table,
           A1_w, A1_b, A1_c, W_enc1, A2_w, A2_b, A2_c, W_enc2,
           gru_w_ih_l0f, gru_w_hh_l0f, gru_b_ih_l0f, gru_b_hh_l0f,
           gru_w_ih_l0b, gru_w_hh_l0b, gru_b_ih_l0b, gru_b_hh_l0b,
           gru_w_ih_l1f, gru_w_hh_l1f, gru_b_ih_l1f, gru_b_hh_l1f,
           gru_w_ih_l1b, gru_w_hh_l1b, gru_b_ih_l1b, gru_b_hh_l1b,
           Af_w, Af_b, Af_c, weight1, weight2):
  feats, xseq = _sc_gather(nodes, neigh_table.reshape(-1),
                           seq_input.reshape(-1), embed_table,
                           seq_embed_table)
  r = lambda v: v.reshape(1, -1)

  def bd(wf, wb):
    z = jnp.zeros((wf.shape[1] + wb.shape[1], 6 * H), jnp.float32)
    z = z.at[:wf.shape[1], :3 * H].set(wf.T).at[wf.shape[1]:, 3 * H:].set(wb.T)
    return z.astype(jnp.bfloat16)

  rep = lambda v, n: jnp.tile(v.reshape(-1, 1), (1, n))
  gweights = (A1_w, r(A1_b), r(A1_c), W_enc1,
              A2_w, r(A2_b), r(A2_c), W_enc2)
  sweights = (
      gru_w_ih_l0f.T, gru_w_ih_l0b.T, r(gru_b_ih_l0f), r(gru_b_ih_l0b),
      bd(gru_w_hh_l0f, gru_w_hh_l0b),
      jnp.concatenate([gru_b_hh_l0f, gru_b_hh_l0b]).reshape(1, -1),
      gru_w_ih_l1f.T, gru_w_ih_l1b.T, r(gru_b_ih_l1f), r(gru_b_ih_l1b),
      bd(gru_w_hh_l1f, gru_w_hh_l1b),
      jnp.concatenate([gru_b_hh_l1f, gru_b_hh_l1b]).reshape(1, -1),
      Af_w, r(Af_b), rep(Af_c, E2), weight1, weight2,
  )
  graph = _graph_call(feats, gweights)
  return _seq_call(xseq, graph, sweights)


# speedup vs baseline: 1.0547x; 1.0547x over previous
import functools

import jax
import jax.numpy as jnp
from jax import lax
from jax.experimental import pallas as pl
from jax.experimental.pallas import tpu as pltpu
from jax.experimental.pallas import tpu_sc as plsc

N_NODES = 100000
D = 128
S1 = 10
S2 = 10
H = 128
ATT = 64
E1 = 128
E2 = 128
B = 1024
L = 50
NC_OUT = 50

NW = 32
BPW = B // NW
N1 = BPW * S2
N2 = N1 * S1
LPW = B * L // NW
RCH = 128


def _sc_gather(nodes, neigh_flat, seq_flat, embed_table, seq_embed_table):
  mesh = plsc.VectorSubcoreMesh(core_axis_name="c", subcore_axis_name="s")

  @functools.partial(
      pl.kernel,
      out_type=(
          jax.ShapeDtypeStruct((B * S2 * S1, D), jnp.float32),
          jax.ShapeDtypeStruct((B * L, D), jnp.float32),
      ),
      mesh=mesh,
      compiler_params=pltpu.CompilerParams(needs_layout_passes=False),
      scratch_types=[
          pltpu.VMEM((BPW,), jnp.int32),
          pltpu.VMEM((N1,), jnp.int32),
          pltpu.VMEM((N1,), jnp.int32),
          pltpu.VMEM((N2,), jnp.int32),
          pltpu.VMEM((N2,), jnp.int32),
          pltpu.VMEM((LPW,), jnp.int32),
          pltpu.VMEM((RCH, D), jnp.float32),
          pltpu.VMEM((RCH, D), jnp.float32),
          pltpu.SemaphoreType.DMA,
          pltpu.SemaphoreType.DMA,
      ],
  )
  def k(nodes_h, nf_h, seq_h, emb_h, semb_h, feats_h, xseq_h,
        nodes_v, idx1_v, ids1_v, idx2_v, ids2_v, seq_v, bufa, bufb,
        sema, semb):
    wid = lax.axis_index("s") * 2 + lax.axis_index("c")
    pltpu.sync_copy(nodes_h.at[pl.ds(wid * BPW, BPW)], nodes_v)
    pltpu.sync_copy(seq_h.at[pl.ds(wid * LPW, LPW)], seq_v)

    def expand_by_10(i, src_ref, dst_ref):
      j = lax.iota(jnp.int32, 16) + i * 16
      m = j // 10
      r = j - m * 10
      vals = plsc.load_gather(src_ref, [m])
      dst_ref[pl.ds(pl.multiple_of(i * 16, 16), 16)] = vals * 10 + r

    lax.fori_loop(0, N1 // 16, lambda i, c: (expand_by_10(i, nodes_v, idx1_v), c)[1], 0)
    for off, n in ((0, 128), (128, 128), (256, 64)):
      pltpu.async_copy(nf_h.at[idx1_v.at[pl.ds(off, n)]],
                       ids1_v.at[pl.ds(off, n)], sema).wait()
    lax.fori_loop(0, N2 // 16, lambda i, c: (expand_by_10(i, ids1_v, idx2_v), c)[1], 0)
    for c in range(N2 // RCH):
      pltpu.async_copy(nf_h.at[idx2_v.at[pl.ds(c * RCH, RCH)]],
                       ids2_v.at[pl.ds(c * RCH, RCH)], sema).wait()

    NFC = N2 // RCH
    NSC = LPW // RCH
    jobs = []
    for c in range(NFC):
      jobs.append((emb_h, ids2_v, c * RCH, RCH, feats_h, wid * N2 + c * RCH))
    for c in range(NSC):
      jobs.append((semb_h, seq_v, c * RCH, RCH, xseq_h, wid * LPW + c * RCH))
    jobs.append((semb_h, seq_v, NSC * RCH, LPW - NSC * RCH, xseq_h,
                 wid * LPW + NSC * RCH))

    bufs = (bufa, bufb)
    sems = (sema, semb)

    def start(j, slot):
      tab, idx, ioff, n, _, _ = jobs[j]
      return pltpu.async_copy(tab.at[idx.at[pl.ds(ioff, n)]],
                              bufs[slot].at[pl.ds(0, n)], sems[slot])

    cp = start(0, 0)
    for j in range(len(jobs)):
      nxt = start(j + 1, (j + 1) % 2) if j + 1 < len(jobs) else None
      cp.wait()
      _, _, _, n, out_h, ooff = jobs[j]
      pltpu.sync_copy(bufs[j % 2].at[pl.ds(0, n)], out_h.at[pl.ds(ooff, n)])
      cp = nxt

  return k(nodes, neigh_flat, seq_flat, embed_table, seq_embed_table)


CH = 128
GRID = B // CH
CHS = 256
GRIDS = B // CHS


def _att_pool(x3, w, b, c):
  a = jnp.tanh(
      lax.dot_general(x3, w, (((2,), (0,)), ((), ())),
                      preferred_element_type=jnp.float32) + b[None, :, :])
  sc = jnp.sum(a * c[None, :, :], axis=2)
  m = jnp.max(sc, axis=1, keepdims=True)
  e = jnp.exp(sc - m)
  p = e * (1.0 / jnp.sum(e, axis=1, keepdims=True))
  return jnp.sum(x3 * p[:, :, None], axis=1)


def _gru_cell(gi, gh, h):
  ir, iz, inn = gi[:, :H], gi[:, H:2 * H], gi[:, 2 * H:]
  hr, hz, hn = gh[:, :H], gh[:, H:2 * H], gh[:, 2 * H:]
  r = jax.nn.sigmoid(ir + hr)
  z = jax.nn.sigmoid(iz + hz)
  n = jnp.tanh(inn + r * hn)
  return (1.0 - z) * n + z * h


def _mm(x, w):
  return lax.dot_general(x, w, (((1,), (0,)), ((), ())),
                         preferred_element_type=jnp.float32)


def _graph_body(feats_ref, a1w, a1b, a1c, w1, a2w, a2b, a2c, w2, out_ref):
  f3 = feats_ref[...].reshape(CH * S2, S1, D)
  ws1 = _att_pool(f3, a1w[...], a1b[...], a1c[...])
  agg1 = jax.nn.relu(ws1 * (1.0 / S1))
  e1 = jax.nn.relu(
      lax.dot_general(agg1, w1[...], (((1,), (1,)), ((), ())),
                      preferred_element_type=jnp.float32))
  e13 = e1.reshape(CH, S2, E1)
  ws2 = _att_pool(e13, a2w[...], a2b[...], a2c[...])
  agg2 = jax.nn.relu(ws2 * (1.0 / S2))
  out_ref[...] = jax.nn.relu(
      lax.dot_general(agg2, w2[...], (((1,), (1,)), ((), ())),
                      preferred_element_type=jnp.float32))


def _seq_body(xseq_ref, graph_ref,
              wih0ft, wih0bt, bih0f, bih0b, bdhh0, bhh0,
              wih1ft, wih1bt, bih1f, bih1b, bdhh1, bhh1,
              afw, afb, afc, wgt1, wgt2, out_ref, xtm, o0f, o0b, gi):
  graph = graph_ref[...]
  x3 = xseq_ref[...].reshape(CHS, L, D)
  for t in range(L):
    xtm[t] = x3[:, t, :]
  TS = 10
  for tb in range(L // TS):
    xs = xtm[tb * TS:(tb + 1) * TS].reshape(TS * CHS, D)
    gi[pl.ds(tb * TS * CHS, TS * CHS), :3 * H] = (_mm(xs, wih0ft[...]) + bih0f[...]).astype(jnp.bfloat16)
    gi[pl.ds(tb * TS * CHS, TS * CHS), 3 * H:] = (_mm(xs, wih0bt[...]) + bih0b[...]).astype(jnp.bfloat16)

  def gru_loop(bd, bh, store):
    def step(t, carry):
      hf, hb = carry
      hcat = jnp.concatenate([hf, hb], axis=1).astype(jnp.bfloat16)
      gh = _mm(hcat, bd) + bh
      gif = gi[pl.ds(t * CHS, CHS), :3 * H].astype(jnp.float32)
      gib = gi[pl.ds((L - 1 - t) * CHS, CHS), 3 * H:].astype(jnp.float32)
      hf = _gru_cell(gif, gh[:, :3 * H], hf)
      hb = _gru_cell(gib, gh[:, 3 * H:], hb)
      if store:
        o0f[t] = hf
        o0b[L - 1 - t] = hb
      return hf, hb

    z2 = jnp.zeros((CHS, H), jnp.float32)
    return lax.fori_loop(0, L, step, (z2, z2), unroll=2)

  gru_loop(bdhh0[...], bhh0[...], True)

  for tb in range(L // TS):
    o01 = jnp.concatenate(
        [o0f[tb * TS:(tb + 1) * TS], o0b[tb * TS:(tb + 1) * TS]],
        axis=2).reshape(TS * CHS, 2 * H)
    gi[pl.ds(tb * TS * CHS, TS * CHS), :3 * H] = (_mm(o01, wih1ft[...]) + bih1f[...]).astype(jnp.bfloat16)
    gi[pl.ds(tb * TS * CHS, TS * CHS), 3 * H:] = (_mm(o01, wih1bt[...]) + bih1b[...]).astype(jnp.bfloat16)
  h1f, h1b = gru_loop(bdhh1[...], bhh1[...], False)
  rnn = jnp.concatenate([h1f, h1b], axis=1)
  rnn_proj = lax.dot_general(rnn, wgt1[...], (((1,), (0,)), ((), ())),
                             preferred_element_type=jnp.float32)

  def att_s(v):
    a = jnp.tanh(_mm(v, afw[...]) + afb[...])
    return _mm(a, afc[...])

  sr = att_s(rnn_proj)
  sg = att_s(graph)
  m = jnp.maximum(sr, sg)
  er = jnp.exp(sr - m)
  eg = jnp.exp(sg - m)
  ws = (er * rnn_proj + eg * graph) / (er + eg)
  out_ref[...] = lax.dot_general(ws, wgt2[...], (((1,), (0,)), ((), ())),
                                 preferred_element_type=jnp.float32)


def _full(a):
  return pl.BlockSpec(a.shape, lambda c: (0,) * a.ndim)


def _graph_call(feats, gweights):
  return pl.pallas_call(
      _graph_body,
      grid=(GRID,),
      in_specs=[pl.BlockSpec((CH * S2 * S1, D), lambda c: (c, 0))]
      + [_full(w) for w in gweights],
      out_specs=pl.BlockSpec((CH, E2), lambda c: (c, 0)),
      out_shape=jax.ShapeDtypeStruct((B, E2), jnp.float32),
      compiler_params=pltpu.CompilerParams(
          dimension_semantics=("arbitrary",),
          vmem_limit_bytes=60000 * 1024,
      ),
  )(feats, *gweights)


def _seq_call(xseq, graph, sweights):
  return pl.pallas_call(
      _seq_body,
      grid=(GRIDS,),
      in_specs=[
          pl.BlockSpec((CHS * L, D), lambda c: (c, 0)),
          pl.BlockSpec((CHS, E2), lambda c: (c, 0)),
      ] + [_full(w) for w in sweights],
      out_specs=pl.BlockSpec((CHS, NC_OUT), lambda c: (c, 0)),
      out_shape=jax.ShapeDtypeStruct((B, NC_OUT), jnp.float32),
      scratch_shapes=[
          pltpu.VMEM((L, CHS, D), jnp.float32),
          pltpu.VMEM((L, CHS, H), jnp.float32),
          pltpu.VMEM((L, CHS, H), jnp.float32),
          pltpu.VMEM((L * CHS, 6 * H), jnp.bfloat16),
      ],
      compiler_params=pltpu.CompilerParams(
          dimension_semantics=("arbitrary",),
          vmem_limit_bytes=60000 * 1024,
      ),
  )(xseq, graph, *sweights)


def kernel(nodes, seq_input, neigh_table, embed_table, seq_embed_table,
           A1_w, A1_b, A1_c, W_enc1, A2_w, A2_b, A2_c, W_enc2,
           gru_w_ih_l0f, gru_w_hh_l0f, gru_b_ih_l0f, gru_b_hh_l0f,
           gru_w_ih_l0b, gru_w_hh_l0b, gru_b_ih_l0b, gru_b_hh_l0b,
           gru_w_ih_l1f, gru_w_hh_l1f, gru_b_ih_l1f, gru_b_hh_l1f,
           gru_w_ih_l1b, gru_w_hh_l1b, gru_b_ih_l1b, gru_b_hh_l1b,
           Af_w, Af_b, Af_c, weight1, weight2):
  feats, xseq = _sc_gather(nodes, neigh_table.reshape(-1),
                           seq_input.reshape(-1), embed_table,
                           seq_embed_table)
  r = lambda v: v.reshape(1, -1)

  def bd(wf, wb):
    z = jnp.zeros((wf.shape[1] + wb.shape[1], 6 * H), jnp.float32)
    z = z.at[:wf.shape[1], :3 * H].set(wf.T).at[wf.shape[1]:, 3 * H:].set(wb.T)
    return z.astype(jnp.bfloat16)

  rep = lambda v, n: jnp.tile(v.reshape(-1, 1), (1, n))
  gweights = (A1_w, r(A1_b), r(A1_c), W_enc1,
              A2_w, r(A2_b), r(A2_c), W_enc2)
  sweights = (
      gru_w_ih_l0f.T, gru_w_ih_l0b.T, r(gru_b_ih_l0f), r(gru_b_ih_l0b),
      bd(gru_w_hh_l0f, gru_w_hh_l0b),
      jnp.concatenate([gru_b_hh_l0f, gru_b_hh_l0b]).reshape(1, -1),
      gru_w_ih_l1f.T, gru_w_ih_l1b.T, r(gru_b_ih_l1f), r(gru_b_ih_l1b),
      bd(gru_w_hh_l1f, gru_w_hh_l1b),
      jnp.concatenate([gru_b_hh_l1f, gru_b_hh_l1b]).reshape(1, -1),
      Af_w, r(Af_b), rep(Af_c, E2), weight1, weight2,
  )
  graph = _graph_call(feats, gweights)
  return _seq_call(xseq, graph, sweights)


# speedup vs baseline: 1.0720x; 1.0164x over previous
import functools

import jax
import jax.numpy as jnp
from jax import lax
from jax.experimental import pallas as pl
from jax.experimental.pallas import tpu as pltpu
from jax.experimental.pallas import tpu_sc as plsc

N_NODES = 100000
D = 128
S1 = 10
S2 = 10
H = 128
ATT = 64
E1 = 128
E2 = 128
B = 1024
L = 50
NC_OUT = 50

NW = 32
BPW = B // NW
N1 = BPW * S2
N2 = N1 * S1
LPW = B * L // NW
RCH = 128


def _sc_gather(nodes, neigh_flat, seq_flat, embed_table, seq_embed_table):
  mesh = plsc.VectorSubcoreMesh(core_axis_name="c", subcore_axis_name="s")

  @functools.partial(
      pl.kernel,
      out_type=(
          jax.ShapeDtypeStruct((B * S2 * S1, D), jnp.float32),
          jax.ShapeDtypeStruct((B * L, D), jnp.float32),
      ),
      mesh=mesh,
      compiler_params=pltpu.CompilerParams(needs_layout_passes=False),
      scratch_types=[
          pltpu.VMEM((BPW,), jnp.int32),
          pltpu.VMEM((N1,), jnp.int32),
          pltpu.VMEM((N1,), jnp.int32),
          pltpu.VMEM((N2,), jnp.int32),
          pltpu.VMEM((N2,), jnp.int32),
          pltpu.VMEM((LPW,), jnp.int32),
          pltpu.VMEM((RCH, D), jnp.float32),
          pltpu.VMEM((RCH, D), jnp.float32),
          pltpu.SemaphoreType.DMA,
          pltpu.SemaphoreType.DMA,
      ],
  )
  def k(nodes_h, nf_h, seq_h, emb_h, semb_h, feats_h, xseq_h,
        nodes_v, idx1_v, ids1_v, idx2_v, ids2_v, seq_v, bufa, bufb,
        sema, semb):
    wid = lax.axis_index("s") * 2 + lax.axis_index("c")
    pltpu.sync_copy(nodes_h.at[pl.ds(wid * BPW, BPW)], nodes_v)
    pltpu.sync_copy(seq_h.at[pl.ds(wid * LPW, LPW)], seq_v)

    def expand_by_10(i, src_ref, dst_ref):
      j = lax.iota(jnp.int32, 16) + i * 16
      m = j // 10
      r = j - m * 10
      vals = plsc.load_gather(src_ref, [m])
      dst_ref[pl.ds(pl.multiple_of(i * 16, 16), 16)] = vals * 10 + r

    lax.fori_loop(0, N1 // 16, lambda i, c: (expand_by_10(i, nodes_v, idx1_v), c)[1], 0)
    for off, n in ((0, 128), (128, 128), (256, 64)):
      pltpu.async_copy(nf_h.at[idx1_v.at[pl.ds(off, n)]],
                       ids1_v.at[pl.ds(off, n)], sema).wait()
    lax.fori_loop(0, N2 // 16, lambda i, c: (expand_by_10(i, ids1_v, idx2_v), c)[1], 0)
    for c in range(N2 // RCH):
      pltpu.async_copy(nf_h.at[idx2_v.at[pl.ds(c * RCH, RCH)]],
                       ids2_v.at[pl.ds(c * RCH, RCH)], sema).wait()

    NFC = N2 // RCH
    NSC = LPW // RCH
    jobs = []
    for c in range(NFC):
      jobs.append((emb_h, ids2_v, c * RCH, RCH, feats_h, wid * N2 + c * RCH))
    for c in range(NSC):
      jobs.append((semb_h, seq_v, c * RCH, RCH, xseq_h, wid * LPW + c * RCH))
    jobs.append((semb_h, seq_v, NSC * RCH, LPW - NSC * RCH, xseq_h,
                 wid * LPW + NSC * RCH))

    bufs = (bufa, bufb)
    sems = (sema, semb)

    def start(j, slot):
      tab, idx, ioff, n, _, _ = jobs[j]
      return pltpu.async_copy(tab.at[idx.at[pl.ds(ioff, n)]],
                              bufs[slot].at[pl.ds(0, n)], sems[slot])

    cp = start(0, 0)
    for j in range(len(jobs)):
      nxt = start(j + 1, (j + 1) % 2) if j + 1 < len(jobs) else None
      cp.wait()
      _, _, _, n, out_h, ooff = jobs[j]
      pltpu.sync_copy(bufs[j % 2].at[pl.ds(0, n)], out_h.at[pl.ds(ooff, n)])
      cp = nxt

  return k(nodes, neigh_flat, seq_flat, embed_table, seq_embed_table)


CH = 128
GRID = B // CH
CHS = 256
GRIDS = B // CHS


def _att_pool(x3, w, b, c):
  a = jnp.tanh(
      lax.dot_general(x3, w, (((2,), (0,)), ((), ())),
                      preferred_element_type=jnp.float32) + b[None, :, :])
  sc = jnp.sum(a * c[None, :, :], axis=2)
  m = jnp.max(sc, axis=1, keepdims=True)
  e = jnp.exp(sc - m)
  p = e * (1.0 / jnp.sum(e, axis=1, keepdims=True))
  return jnp.sum(x3 * p[:, :, None], axis=1)


def _gru_cell(gi, gh, h):
  ir, iz, inn = gi[:, :H], gi[:, H:2 * H], gi[:, 2 * H:]
  hr, hz, hn = gh[:, :H], gh[:, H:2 * H], gh[:, 2 * H:]
  r = jax.nn.sigmoid(ir + hr)
  z = jax.nn.sigmoid(iz + hz)
  n = jnp.tanh(inn + r * hn)
  return (1.0 - z) * n + z * h


def _mm(x, w):
  return lax.dot_general(x, w, (((1,), (0,)), ((), ())),
                         preferred_element_type=jnp.float32)


def _graph_body(feats_ref, a1w, a1b, a1c, w1, a2w, a2b, a2c, w2, out_ref):
  f3 = feats_ref[...].reshape(CH * S2, S1, D)
  ws1 = _att_pool(f3, a1w[...], a1b[...], a1c[...])
  agg1 = jax.nn.relu(ws1 * (1.0 / S1))
  e1 = jax.nn.relu(
      lax.dot_general(agg1, w1[...], (((1,), (1,)), ((), ())),
                      preferred_element_type=jnp.float32))
  e13 = e1.reshape(CH, S2, E1)
  ws2 = _att_pool(e13, a2w[...], a2b[...], a2c[...])
  agg2 = jax.nn.relu(ws2 * (1.0 / S2))
  out_ref[...] = jax.nn.relu(
      lax.dot_general(agg2, w2[...], (((1,), (1,)), ((), ())),
                      preferred_element_type=jnp.float32))


def _seq_body(xseq_ref, graph_ref,
              wih0ft, wih0bt, bih0f, bih0b, bdhh0, bhh0,
              wih1ft, wih1bt, bih1f, bih1b, bdhh1, bhh1,
              afw, afb, afc, wgt1, wgt2, out_ref, xtm, o0f, o0b, gi):
  graph = graph_ref[...]
  x3 = xseq_ref[...].reshape(CHS, L, D)
  for t in range(L):
    xtm[t] = x3[:, t, :]
  TS = 10
  for tb in range(L // TS):
    xs = xtm[tb * TS:(tb + 1) * TS].reshape(TS * CHS, D)
    gi[pl.ds(tb * TS * CHS, TS * CHS), :3 * H] = (_mm(xs, wih0ft[...]) + bih0f[...]).astype(jnp.bfloat16)
    gi[pl.ds(tb * TS * CHS, TS * CHS), 3 * H:] = (_mm(xs, wih0bt[...]) + bih0b[...]).astype(jnp.bfloat16)

  def gru_loop(bd, bh, store):
    def step(t, carry):
      hf, hb = carry
      hcat = jnp.concatenate([hf, hb], axis=1).astype(jnp.bfloat16)
      gh = _mm(hcat, bd) + bh
      gif = gi[pl.ds(t * CHS, CHS), :3 * H].astype(jnp.float32)
      gib = gi[pl.ds((L - 1 - t) * CHS, CHS), 3 * H:].astype(jnp.float32)
      hf = _gru_cell(gif, gh[:, :3 * H], hf)
      hb = _gru_cell(gib, gh[:, 3 * H:], hb)
      if store:
        o0f[t] = hf
        o0b[L - 1 - t] = hb
      return hf, hb

    z2 = jnp.zeros((CHS, H), jnp.float32)
    return lax.fori_loop(0, L, step, (z2, z2), unroll=5)

  gru_loop(bdhh0[...], bhh0[...], True)

  for tb in range(L // TS):
    o01 = jnp.concatenate(
        [o0f[tb * TS:(tb + 1) * TS], o0b[tb * TS:(tb + 1) * TS]],
        axis=2).reshape(TS * CHS, 2 * H)
    gi[pl.ds(tb * TS * CHS, TS * CHS), :3 * H] = (_mm(o01, wih1ft[...]) + bih1f[...]).astype(jnp.bfloat16)
    gi[pl.ds(tb * TS * CHS, TS * CHS), 3 * H:] = (_mm(o01, wih1bt[...]) + bih1b[...]).astype(jnp.bfloat16)
  h1f, h1b = gru_loop(bdhh1[...], bhh1[...], False)
  rnn = jnp.concatenate([h1f, h1b], axis=1)
  rnn_proj = lax.dot_general(rnn, wgt1[...], (((1,), (0,)), ((), ())),
                             preferred_element_type=jnp.float32)

  def att_s(v):
    a = jnp.tanh(_mm(v, afw[...]) + afb[...])
    return _mm(a, afc[...])

  sr = att_s(rnn_proj)
  sg = att_s(graph)
  m = jnp.maximum(sr, sg)
  er = jnp.exp(sr - m)
  eg = jnp.exp(sg - m)
  ws = (er * rnn_proj + eg * graph) / (er + eg)
  out_ref[...] = lax.dot_general(ws, wgt2[...], (((1,), (0,)), ((), ())),
                                 preferred_element_type=jnp.float32)


def _full(a):
  return pl.BlockSpec(a.shape, lambda c: (0,) * a.ndim)


def _graph_call(feats, gweights):
  return pl.pallas_call(
      _graph_body,
      grid=(GRID,),
      in_specs=[pl.BlockSpec((CH * S2 * S1, D), lambda c: (c, 0))]
      + [_full(w) for w in gweights],
      out_specs=pl.BlockSpec((CH, E2), lambda c: (c, 0)),
      out_shape=jax.ShapeDtypeStruct((B, E2), jnp.float32),
      compiler_params=pltpu.CompilerParams(
          dimension_semantics=("arbitrary",),
          vmem_limit_bytes=60000 * 1024,
      ),
  )(feats, *gweights)


def _seq_call(xseq, graph, sweights):
  return pl.pallas_call(
      _seq_body,
      grid=(GRIDS,),
      in_specs=[
          pl.BlockSpec((CHS * L, D), lambda c: (c, 0)),
          pl.BlockSpec((CHS, E2), lambda c: (c, 0)),
      ] + [_full(w) for w in sweights],
      out_specs=pl.BlockSpec((CHS, NC_OUT), lambda c: (c, 0)),
      out_shape=jax.ShapeDtypeStruct((B, NC_OUT), jnp.float32),
      scratch_shapes=[
          pltpu.VMEM((L, CHS, D), jnp.float32),
          pltpu.VMEM((L, CHS, H), jnp.float32),
          pltpu.VMEM((L, CHS, H), jnp.float32),
          pltpu.VMEM((L * CHS, 6 * H), jnp.bfloat16),
      ],
      compiler_params=pltpu.CompilerParams(
          dimension_semantics=("arbitrary",),
          vmem_limit_bytes=60000 * 1024,
      ),
  )(xseq, graph, *sweights)


def kernel(nodes, seq_input, neigh_table, embed_table, seq_embed_table,
           A1_w, A1_b, A1_c, W_enc1, A2_w, A2_b, A2_c, W_enc2,
           gru_w_ih_l0f, gru_w_hh_l0f, gru_b_ih_l0f, gru_b_hh_l0f,
           gru_w_ih_l0b, gru_w_hh_l0b, gru_b_ih_l0b, gru_b_hh_l0b,
           gru_w_ih_l1f, gru_w_hh_l1f, gru_b_ih_l1f, gru_b_hh_l1f,
           gru_w_ih_l1b, gru_w_hh_l1b, gru_b_ih_l1b, gru_b_hh_l1b,
           Af_w, Af_b, Af_c, weight1, weight2):
  feats, xseq = _sc_gather(nodes, neigh_table.reshape(-1),
                           seq_input.reshape(-1), embed_table,
                           seq_embed_table)
  r = lambda v: v.reshape(1, -1)

  def bd(wf, wb):
    z = jnp.zeros((wf.shape[1] + wb.shape[1], 6 * H), jnp.float32)
    z = z.at[:wf.shape[1], :3 * H].set(wf.T).at[wf.shape[1]:, 3 * H:].set(wb.T)
    return z.astype(jnp.bfloat16)

  rep = lambda v, n: jnp.tile(v.reshape(-1, 1), (1, n))
  gweights = (A1_w, r(A1_b), r(A1_c), W_enc1,
              A2_w, r(A2_b), r(A2_c), W_enc2)
  sweights = (
      gru_w_ih_l0f.T, gru_w_ih_l0b.T, r(gru_b_ih_l0f), r(gru_b_ih_l0b),
      bd(gru_w_hh_l0f, gru_w_hh_l0b),
      jnp.concatenate([gru_b_hh_l0f, gru_b_hh_l0b]).reshape(1, -1),
      gru_w_ih_l1f.T, gru_w_ih_l1b.T, r(gru_b_ih_l1f), r(gru_b_ih_l1b),
      bd(gru_w_hh_l1f, gru_w_hh_l1b),
      jnp.concatenate([gru_b_hh_l1f, gru_b_hh_l1b]).reshape(1, -1),
      Af_w, r(Af_b), rep(Af_c, E2), weight1, weight2,
  )
  graph = _graph_call(feats, gweights)
  return _seq_call(xseq, graph, sweights)


# speedup vs baseline: 1.0807x; 1.0081x over previous
import functools

import jax
import jax.numpy as jnp
from jax import lax
from jax.experimental import pallas as pl
from jax.experimental.pallas import tpu as pltpu
from jax.experimental.pallas import tpu_sc as plsc

N_NODES = 100000
D = 128
S1 = 10
S2 = 10
H = 128
ATT = 64
E1 = 128
E2 = 128
B = 1024
L = 50
NC_OUT = 50

NW = 32
BPW = B // NW
N1 = BPW * S2
N2 = N1 * S1
LPW = B * L // NW
RCH = 128


def _sc_gather(nodes, neigh_flat, seq_flat, embed_table, seq_embed_table):
  mesh = plsc.VectorSubcoreMesh(core_axis_name="c", subcore_axis_name="s")

  @functools.partial(
      pl.kernel,
      out_type=(
          jax.ShapeDtypeStruct((B * S2 * S1, D), jnp.float32),
          jax.ShapeDtypeStruct((B * L, D), jnp.float32),
      ),
      mesh=mesh,
      compiler_params=pltpu.CompilerParams(needs_layout_passes=False),
      scratch_types=[
          pltpu.VMEM((BPW,), jnp.int32),
          pltpu.VMEM((N1,), jnp.int32),
          pltpu.VMEM((N1,), jnp.int32),
          pltpu.VMEM((N2,), jnp.int32),
          pltpu.VMEM((N2,), jnp.int32),
          pltpu.VMEM((LPW,), jnp.int32),
          pltpu.VMEM((RCH, D), jnp.float32),
          pltpu.VMEM((RCH, D), jnp.float32),
          pltpu.SemaphoreType.DMA,
          pltpu.SemaphoreType.DMA,
      ],
  )
  def k(nodes_h, nf_h, seq_h, emb_h, semb_h, feats_h, xseq_h,
        nodes_v, idx1_v, ids1_v, idx2_v, ids2_v, seq_v, bufa, bufb,
        sema, semb):
    wid = lax.axis_index("s") * 2 + lax.axis_index("c")
    pltpu.sync_copy(nodes_h.at[pl.ds(wid * BPW, BPW)], nodes_v)
    pltpu.sync_copy(seq_h.at[pl.ds(wid * LPW, LPW)], seq_v)

    def expand_by_10(i, src_ref, dst_ref):
      j = lax.iota(jnp.int32, 16) + i * 16
      m = j // 10
      r = j - m * 10
      vals = plsc.load_gather(src_ref, [m])
      dst_ref[pl.ds(pl.multiple_of(i * 16, 16), 16)] = vals * 10 + r

    lax.fori_loop(0, N1 // 16, lambda i, c: (expand_by_10(i, nodes_v, idx1_v), c)[1], 0)
    for off, n in ((0, 128), (128, 128), (256, 64)):
      pltpu.async_copy(nf_h.at[idx1_v.at[pl.ds(off, n)]],
                       ids1_v.at[pl.ds(off, n)], sema).wait()
    lax.fori_loop(0, N2 // 16, lambda i, c: (expand_by_10(i, ids1_v, idx2_v), c)[1], 0)
    for c in range(N2 // RCH):
      pltpu.async_copy(nf_h.at[idx2_v.at[pl.ds(c * RCH, RCH)]],
                       ids2_v.at[pl.ds(c * RCH, RCH)], sema).wait()

    NFC = N2 // RCH
    NSC = LPW // RCH
    jobs = []
    for c in range(NFC):
      jobs.append((emb_h, ids2_v, c * RCH, RCH, feats_h, wid * N2 + c * RCH))
    for c in range(NSC):
      jobs.append((semb_h, seq_v, c * RCH, RCH, xseq_h, wid * LPW + c * RCH))
    jobs.append((semb_h, seq_v, NSC * RCH, LPW - NSC * RCH, xseq_h,
                 wid * LPW + NSC * RCH))

    bufs = (bufa, bufb)
    sems = (sema, semb)

    def start(j, slot):
      tab, idx, ioff, n, _, _ = jobs[j]
      return pltpu.async_copy(tab.at[idx.at[pl.ds(ioff, n)]],
                              bufs[slot].at[pl.ds(0, n)], sems[slot])

    cp = start(0, 0)
    for j in range(len(jobs)):
      nxt = start(j + 1, (j + 1) % 2) if j + 1 < len(jobs) else None
      cp.wait()
      _, _, _, n, out_h, ooff = jobs[j]
      pltpu.sync_copy(bufs[j % 2].at[pl.ds(0, n)], out_h.at[pl.ds(ooff, n)])
      cp = nxt

  return k(nodes, neigh_flat, seq_flat, embed_table, seq_embed_table)


CH = 128
GRID = B // CH
CHS = 256
GRIDS = B // CHS


def _att_pool(x3, w, b, c):
  a = jnp.tanh(
      lax.dot_general(x3, w, (((2,), (0,)), ((), ())),
                      preferred_element_type=jnp.float32) + b[None, :, :])
  sc = jnp.sum(a * c[None, :, :], axis=2)
  m = jnp.max(sc, axis=1, keepdims=True)
  e = jnp.exp(sc - m)
  p = e * (1.0 / jnp.sum(e, axis=1, keepdims=True))
  return jnp.sum(x3 * p[:, :, None], axis=1)


def _gru_cell(gi, gh, h):
  ir, iz, inn = gi[:, :H], gi[:, H:2 * H], gi[:, 2 * H:]
  hr, hz, hn = gh[:, :H], gh[:, H:2 * H], gh[:, 2 * H:]
  r = jax.nn.sigmoid(ir + hr)
  z = jax.nn.sigmoid(iz + hz)
  n = jnp.tanh(inn + r * hn)
  return (1.0 - z) * n + z * h


def _mm(x, w):
  return lax.dot_general(x, w, (((1,), (0,)), ((), ())),
                         preferred_element_type=jnp.float32)


def _graph_body(feats_ref, a1w, a1b, a1c, w1, a2w, a2b, a2c, w2, out_ref):
  f3 = feats_ref[...].reshape(CH * S2, S1, D)
  ws1 = _att_pool(f3, a1w[...], a1b[...], a1c[...])
  agg1 = jax.nn.relu(ws1 * (1.0 / S1))
  e1 = jax.nn.relu(
      lax.dot_general(agg1, w1[...], (((1,), (1,)), ((), ())),
                      preferred_element_type=jnp.float32))
  e13 = e1.reshape(CH, S2, E1)
  ws2 = _att_pool(e13, a2w[...], a2b[...], a2c[...])
  agg2 = jax.nn.relu(ws2 * (1.0 / S2))
  out_ref[...] = jax.nn.relu(
      lax.dot_general(agg2, w2[...], (((1,), (1,)), ((), ())),
                      preferred_element_type=jnp.float32))


def _seq_body(xseq_ref, graph_ref,
              wih0ft, wih0bt, bih0f, bih0b, bdhh0, bhh0,
              wih1ft, wih1bt, bih1f, bih1b, bdhh1, bhh1,
              afw, afb, afc, wgt1, wgt2, out_ref, xtm, o0f, o0b, gi):
  graph = graph_ref[...]
  x3 = xseq_ref[...].reshape(CHS, L, D)
  for t in range(L):
    xtm[t] = x3[:, t, :]
  TS = 10
  for tb in range(L // TS):
    xs = xtm[tb * TS:(tb + 1) * TS].reshape(TS * CHS, D)
    gi[pl.ds(tb * TS * CHS, TS * CHS), :3 * H] = (_mm(xs, wih0ft[...]) + bih0f[...]).astype(jnp.bfloat16)
    gi[pl.ds(tb * TS * CHS, TS * CHS), 3 * H:] = (_mm(xs, wih0bt[...]) + bih0b[...]).astype(jnp.bfloat16)

  def gru_loop(bd, bh, store):
    def step(t, carry):
      hf, hb = carry
      hcat = jnp.concatenate([hf, hb], axis=1).astype(jnp.bfloat16)
      gh = _mm(hcat, bd) + bh
      gif = gi[pl.ds(t * CHS, CHS), :3 * H].astype(jnp.float32)
      gib = gi[pl.ds((L - 1 - t) * CHS, CHS), 3 * H:].astype(jnp.float32)
      hf = _gru_cell(gif, gh[:, :3 * H], hf)
      hb = _gru_cell(gib, gh[:, 3 * H:], hb)
      if store:
        o0f[t] = hf
        o0b[L - 1 - t] = hb
      return hf, hb

    z2 = jnp.zeros((CHS, H), jnp.float32)
    return lax.fori_loop(0, L, step, (z2, z2), unroll=10)

  gru_loop(bdhh0[...], bhh0[...], True)

  for tb in range(L // TS):
    o01 = jnp.concatenate(
        [o0f[tb * TS:(tb + 1) * TS], o0b[tb * TS:(tb + 1) * TS]],
        axis=2).reshape(TS * CHS, 2 * H)
    gi[pl.ds(tb * TS * CHS, TS * CHS), :3 * H] = (_mm(o01, wih1ft[...]) + bih1f[...]).astype(jnp.bfloat16)
    gi[pl.ds(tb * TS * CHS, TS * CHS), 3 * H:] = (_mm(o01, wih1bt[...]) + bih1b[...]).astype(jnp.bfloat16)
  h1f, h1b = gru_loop(bdhh1[...], bhh1[...], False)
  rnn = jnp.concatenate([h1f, h1b], axis=1)
  rnn_proj = lax.dot_general(rnn, wgt1[...], (((1,), (0,)), ((), ())),
                             preferred_element_type=jnp.float32)

  def att_s(v):
    a = jnp.tanh(_mm(v, afw[...]) + afb[...])
    return _mm(a, afc[...])

  sr = att_s(rnn_proj)
  sg = att_s(graph)
  m = jnp.maximum(sr, sg)
  er = jnp.exp(sr - m)
  eg = jnp.exp(sg - m)
  ws = (er * rnn_proj + eg * graph) / (er + eg)
  out_ref[...] = lax.dot_general(ws, wgt2[...], (((1,), (0,)), ((), ())),
                                 preferred_element_type=jnp.float32)


def _full(a):
  return pl.BlockSpec(a.shape, lambda c: (0,) * a.ndim)


def _graph_call(feats, gweights):
  return pl.pallas_call(
      _graph_body,
      grid=(GRID,),
      in_specs=[pl.BlockSpec((CH * S2 * S1, D), lambda c: (c, 0))]
      + [_full(w) for w in gweights],
      out_specs=pl.BlockSpec((CH, E2), lambda c: (c, 0)),
      out_shape=jax.ShapeDtypeStruct((B, E2), jnp.float32),
      compiler_params=pltpu.CompilerParams(
          dimension_semantics=("arbitrary",),
          vmem_limit_bytes=60000 * 1024,
      ),
  )(feats, *gweights)


def _seq_call(xseq, graph, sweights):
  return pl.pallas_call(
      _seq_body,
      grid=(GRIDS,),
      in_specs=[
          pl.BlockSpec((CHS * L, D), lambda c: (c, 0)),
          pl.BlockSpec((CHS, E2), lambda c: (c, 0)),
      ] + [_full(w) for w in sweights],
      out_specs=pl.BlockSpec((CHS, NC_OUT), lambda c: (c, 0)),
      out_shape=jax.ShapeDtypeStruct((B, NC_OUT), jnp.float32),
      scratch_shapes=[
          pltpu.VMEM((L, CHS, D), jnp.float32),
          pltpu.VMEM((L, CHS, H), jnp.float32),
          pltpu.VMEM((L, CHS, H), jnp.float32),
          pltpu.VMEM((L * CHS, 6 * H), jnp.bfloat16),
      ],
      compiler_params=pltpu.CompilerParams(
          dimension_semantics=("arbitrary",),
          vmem_limit_bytes=60000 * 1024,
      ),
  )(xseq, graph, *sweights)


def kernel(nodes, seq_input, neigh_table, embed_table, seq_embed_table,
           A1_w, A1_b, A1_c, W_enc1, A2_w, A2_b, A2_c, W_enc2,
           gru_w_ih_l0f, gru_w_hh_l0f, gru_b_ih_l0f, gru_b_hh_l0f,
           gru_w_ih_l0b, gru_w_hh_l0b, gru_b_ih_l0b, gru_b_hh_l0b,
           gru_w_ih_l1f, gru_w_hh_l1f, gru_b_ih_l1f, gru_b_hh_l1f,
           gru_w_ih_l1b, gru_w_hh_l1b, gru_b_ih_l1b, gru_b_hh_l1b,
           Af_w, Af_b, Af_c, weight1, weight2):
  feats, xseq = _sc_gather(nodes, neigh_table.reshape(-1),
                           seq_input.reshape(-1), embed_table,
                           seq_embed_table)
  r = lambda v: v.reshape(1, -1)

  def bd(wf, wb):
    z = jnp.zeros((wf.shape[1] + wb.shape[1], 6 * H), jnp.float32)
    z = z.at[:wf.shape[1], :3 * H].set(wf.T).at[wf.shape[1]:, 3 * H:].set(wb.T)
    return z.astype(jnp.bfloat16)

  rep = lambda v, n: jnp.tile(v.reshape(-1, 1), (1, n))
  gweights = (A1_w, r(A1_b), r(A1_c), W_enc1,
              A2_w, r(A2_b), r(A2_c), W_enc2)
  sweights = (
      gru_w_ih_l0f.T, gru_w_ih_l0b.T, r(gru_b_ih_l0f), r(gru_b_ih_l0b),
      bd(gru_w_hh_l0f, gru_w_hh_l0b),
      jnp.concatenate([gru_b_hh_l0f, gru_b_hh_l0b]).reshape(1, -1),
      gru_w_ih_l1f.T, gru_w_ih_l1b.T, r(gru_b_ih_l1f), r(gru_b_ih_l1b),
      bd(gru_w_hh_l1f, gru_w_hh_l1b),
      jnp.concatenate([gru_b_hh_l1f, gru_b_hh_l1b]).reshape(1, -1),
      Af_w, r(Af_b), rep(Af_c, E2), weight1, weight2,
  )
  graph = _graph_call(feats, gweights)
  return _seq_call(xseq, graph, sweights)


# speedup vs baseline: 1.1468x; 1.0612x over previous
import functools

import jax
import jax.numpy as jnp
from jax import lax
from jax.experimental import pallas as pl
from jax.experimental.pallas import tpu as pltpu
from jax.experimental.pallas import tpu_sc as plsc

N_NODES = 100000
D = 128
S1 = 10
S2 = 10
H = 128
ATT = 64
E1 = 128
E2 = 128
B = 1024
L = 50
NC_OUT = 50

NW = 32
BPW = B // NW
N1 = BPW * S2
N2 = N1 * S1
LPW = B * L // NW
RCH = 128


def _run_jobs(jobs, bufa, bufb, sema, semb):
  bufs = (bufa, bufb)
  sems = (sema, semb)

  def start(j, slot):
    tab, idx, ioff, n, _, _ = jobs[j]
    return pltpu.async_copy(tab.at[idx.at[pl.ds(ioff, n)]],
                            bufs[slot].at[pl.ds(0, n)], sems[slot])

  cp = start(0, 0)
  for j in range(len(jobs)):
    nxt = start(j + 1, (j + 1) % 2) if j + 1 < len(jobs) else None
    cp.wait()
    _, _, _, n, out_h, ooff = jobs[j]
    pltpu.sync_copy(bufs[j % 2].at[pl.ds(0, n)], out_h.at[pl.ds(ooff, n)])
    cp = nxt


def _sc_feats(nodes, neigh_flat, embed_table):
  mesh = plsc.VectorSubcoreMesh(core_axis_name="c", subcore_axis_name="s")

  @functools.partial(
      pl.kernel,
      out_type=jax.ShapeDtypeStruct((B * S2 * S1, D), jnp.float32),
      mesh=mesh,
      compiler_params=pltpu.CompilerParams(needs_layout_passes=False),
      scratch_types=[
          pltpu.VMEM((BPW,), jnp.int32),
          pltpu.VMEM((N1,), jnp.int32),
          pltpu.VMEM((N1,), jnp.int32),
          pltpu.VMEM((N2,), jnp.int32),
          pltpu.VMEM((N2,), jnp.int32),
          pltpu.VMEM((RCH, D), jnp.float32),
          pltpu.VMEM((RCH, D), jnp.float32),
          pltpu.SemaphoreType.DMA,
          pltpu.SemaphoreType.DMA,
      ],
  )
  def k(nodes_h, nf_h, emb_h, feats_h,
        nodes_v, idx1_v, ids1_v, idx2_v, ids2_v, bufa, bufb, sema, semb):
    wid = lax.axis_index("s") * 2 + lax.axis_index("c")
    pltpu.sync_copy(nodes_h.at[pl.ds(wid * BPW, BPW)], nodes_v)

    def expand_by_10(i, src_ref, dst_ref):
      j = lax.iota(jnp.int32, 16) + i * 16
      m = j // 10
      r = j - m * 10
      vals = plsc.load_gather(src_ref, [m])
      dst_ref[pl.ds(pl.multiple_of(i * 16, 16), 16)] = vals * 10 + r

    lax.fori_loop(0, N1 // 16, lambda i, c: (expand_by_10(i, nodes_v, idx1_v), c)[1], 0)
    for off, n in ((0, 128), (128, 128), (256, 64)):
      pltpu.async_copy(nf_h.at[idx1_v.at[pl.ds(off, n)]],
                       ids1_v.at[pl.ds(off, n)], sema).wait()
    lax.fori_loop(0, N2 // 16, lambda i, c: (expand_by_10(i, ids1_v, idx2_v), c)[1], 0)
    for c in range(N2 // RCH):
      pltpu.async_copy(nf_h.at[idx2_v.at[pl.ds(c * RCH, RCH)]],
                       ids2_v.at[pl.ds(c * RCH, RCH)], sema).wait()

    jobs = [(emb_h, ids2_v, c * RCH, RCH, feats_h, wid * N2 + c * RCH)
            for c in range(N2 // RCH)]
    _run_jobs(jobs, bufa, bufb, sema, semb)

  return k(nodes, neigh_flat, embed_table)


def _sc_seq(seq_flat, seq_embed_table):
  mesh = plsc.VectorSubcoreMesh(core_axis_name="c", subcore_axis_name="s")

  @functools.partial(
      pl.kernel,
      out_type=jax.ShapeDtypeStruct((B * L, D), jnp.float32),
      mesh=mesh,
      compiler_params=pltpu.CompilerParams(needs_layout_passes=False),
      scratch_types=[
          pltpu.VMEM((LPW,), jnp.int32),
          pltpu.VMEM((RCH, D), jnp.float32),
          pltpu.VMEM((RCH, D), jnp.float32),
          pltpu.SemaphoreType.DMA,
          pltpu.SemaphoreType.DMA,
      ],
  )
  def k(seq_h, semb_h, xseq_h, seq_v, bufa, bufb, sema, semb):
    wid = lax.axis_index("s") * 2 + lax.axis_index("c")
    pltpu.sync_copy(seq_h.at[pl.ds(wid * LPW, LPW)], seq_v)
    NSC = LPW // RCH
    jobs = [(semb_h, seq_v, c * RCH, RCH, xseq_h, wid * LPW + c * RCH)
            for c in range(NSC)]
    jobs.append((semb_h, seq_v, NSC * RCH, LPW - NSC * RCH, xseq_h,
                 wid * LPW + NSC * RCH))
    _run_jobs(jobs, bufa, bufb, sema, semb)

  return k(seq_flat, seq_embed_table)


CH = 128
GRID = B // CH
CHS = 256
GRIDS = B // CHS


def _att_pool(x3, w, b, c):
  a = jnp.tanh(
      lax.dot_general(x3, w, (((2,), (0,)), ((), ())),
                      preferred_element_type=jnp.float32) + b[None, :, :])
  sc = jnp.sum(a * c[None, :, :], axis=2)
  m = jnp.max(sc, axis=1, keepdims=True)
  e = jnp.exp(sc - m)
  p = e * (1.0 / jnp.sum(e, axis=1, keepdims=True))
  return jnp.sum(x3 * p[:, :, None], axis=1)


def _gru_cell(gi, gh, h):
  ir, iz, inn = gi[:, :H], gi[:, H:2 * H], gi[:, 2 * H:]
  hr, hz, hn = gh[:, :H], gh[:, H:2 * H], gh[:, 2 * H:]
  r = jax.nn.sigmoid(ir + hr)
  z = jax.nn.sigmoid(iz + hz)
  n = jnp.tanh(inn + r * hn)
  return (1.0 - z) * n + z * h


def _mm(x, w):
  return lax.dot_general(x, w, (((1,), (0,)), ((), ())),
                         preferred_element_type=jnp.float32)


def _graph_body(feats_ref, rnn_ref, a1w, a1b, a1c, w1, a2w, a2b, a2c, w2,
                afw, afb, afc, wgt2, out_ref):
  f3 = feats_ref[...].reshape(CH * S2, S1, D)
  ws1 = _att_pool(f3, a1w[...], a1b[...], a1c[...])
  agg1 = jax.nn.relu(ws1 * (1.0 / S1))
  e1 = jax.nn.relu(
      lax.dot_general(agg1, w1[...], (((1,), (1,)), ((), ())),
                      preferred_element_type=jnp.float32))
  e13 = e1.reshape(CH, S2, E1)
  ws2 = _att_pool(e13, a2w[...], a2b[...], a2c[...])
  agg2 = jax.nn.relu(ws2 * (1.0 / S2))
  graph = jax.nn.relu(
      lax.dot_general(agg2, w2[...], (((1,), (1,)), ((), ())),
                      preferred_element_type=jnp.float32))

  rnn_proj = rnn_ref[...]

  def att_s(v):
    a = jnp.tanh(_mm(v, afw[...]) + afb[...])
    return _mm(a, afc[...])

  sr = att_s(rnn_proj)
  sg = att_s(graph)
  m = jnp.maximum(sr, sg)
  er = jnp.exp(sr - m)
  eg = jnp.exp(sg - m)
  ws = (er * rnn_proj + eg * graph) / (er + eg)
  out_ref[...] = lax.dot_general(ws, wgt2[...], (((1,), (0,)), ((), ())),
                                 preferred_element_type=jnp.float32)


def _seq_body(xseq_ref,
              wih0ft, wih0bt, bih0f, bih0b, bdhh0, bhh0,
              wih1ft, wih1bt, bih1f, bih1b, bdhh1, bhh1,
              wgt1, out_ref, xtm, o0f, o0b, gi):
  x3 = xseq_ref[...].reshape(CHS, L, D)
  for t in range(L):
    xtm[t] = x3[:, t, :]
  TS = 10
  for tb in range(L // TS):
    xs = xtm[tb * TS:(tb + 1) * TS].reshape(TS * CHS, D)
    gi[pl.ds(tb * TS * CHS, TS * CHS), :3 * H] = (_mm(xs, wih0ft[...]) + bih0f[...]).astype(jnp.bfloat16)
    gi[pl.ds(tb * TS * CHS, TS * CHS), 3 * H:] = (_mm(xs, wih0bt[...]) + bih0b[...]).astype(jnp.bfloat16)

  def gru_loop(bd, bh, store):
    def step(t, carry):
      hf, hb = carry
      hcat = jnp.concatenate([hf, hb], axis=1).astype(jnp.bfloat16)
      gh = _mm(hcat, bd) + bh
      gif = gi[pl.ds(t * CHS, CHS), :3 * H].astype(jnp.float32)
      gib = gi[pl.ds((L - 1 - t) * CHS, CHS), 3 * H:].astype(jnp.float32)
      hf = _gru_cell(gif, gh[:, :3 * H], hf)
      hb = _gru_cell(gib, gh[:, 3 * H:], hb)
      if store:
        o0f[t] = hf
        o0b[L - 1 - t] = hb
      return hf, hb

    z2 = jnp.zeros((CHS, H), jnp.float32)
    return lax.fori_loop(0, L, step, (z2, z2), unroll=10)

  gru_loop(bdhh0[...], bhh0[...], True)

  for tb in range(L // TS):
    o01 = jnp.concatenate(
        [o0f[tb * TS:(tb + 1) * TS], o0b[tb * TS:(tb + 1) * TS]],
        axis=2).reshape(TS * CHS, 2 * H)
    gi[pl.ds(tb * TS * CHS, TS * CHS), :3 * H] = (_mm(o01, wih1ft[...]) + bih1f[...]).astype(jnp.bfloat16)
    gi[pl.ds(tb * TS * CHS, TS * CHS), 3 * H:] = (_mm(o01, wih1bt[...]) + bih1b[...]).astype(jnp.bfloat16)
  h1f, h1b = gru_loop(bdhh1[...], bhh1[...], False)
  rnn = jnp.concatenate([h1f, h1b], axis=1)
  out_ref[...] = lax.dot_general(rnn, wgt1[...], (((1,), (0,)), ((), ())),
                                 preferred_element_type=jnp.float32)


def _full(a):
  return pl.BlockSpec(a.shape, lambda c: (0,) * a.ndim)


def _graph_call(feats, rnn, gweights):
  return pl.pallas_call(
      _graph_body,
      grid=(GRID,),
      in_specs=[pl.BlockSpec((CH * S2 * S1, D), lambda c: (c, 0)),
                pl.BlockSpec((CH, E2), lambda c: (c, 0))]
      + [_full(w) for w in gweights],
      out_specs=pl.BlockSpec((CH, NC_OUT), lambda c: (c, 0)),
      out_shape=jax.ShapeDtypeStruct((B, NC_OUT), jnp.float32),
      compiler_params=pltpu.CompilerParams(
          dimension_semantics=("arbitrary",),
          vmem_limit_bytes=60000 * 1024,
      ),
  )(feats, rnn, *gweights)


def _seq_call(xseq, sweights):
  return pl.pallas_call(
      _seq_body,
      grid=(GRIDS,),
      in_specs=[
          pl.BlockSpec((CHS * L, D), lambda c: (c, 0)),
      ] + [_full(w) for w in sweights],
      out_specs=pl.BlockSpec((CHS, E2), lambda c: (c, 0)),
      out_shape=jax.ShapeDtypeStruct((B, E2), jnp.float32),
      scratch_shapes=[
          pltpu.VMEM((L, CHS, D), jnp.float32),
          pltpu.VMEM((L, CHS, H), jnp.float32),
          pltpu.VMEM((L, CHS, H), jnp.float32),
          pltpu.VMEM((L * CHS, 6 * H), jnp.bfloat16),
      ],
      compiler_params=pltpu.CompilerParams(
          dimension_semantics=("arbitrary",),
          vmem_limit_bytes=60000 * 1024,
      ),
  )(xseq, *sweights)


def kernel(nodes, seq_input, neigh_table, embed_table, seq_embed_table,
           A1_w, A1_b, A1_c, W_enc1, A2_w, A2_b, A2_c, W_enc2,
           gru_w_ih_l0f, gru_w_hh_l0f, gru_b_ih_l0f, gru_b_hh_l0f,
           gru_w_ih_l0b, gru_w_hh_l0b, gru_b_ih_l0b, gru_b_hh_l0b,
           gru_w_ih_l1f, gru_w_hh_l1f, gru_b_ih_l1f, gru_b_hh_l1f,
           gru_w_ih_l1b, gru_w_hh_l1b, gru_b_ih_l1b, gru_b_hh_l1b,
           Af_w, Af_b, Af_c, weight1, weight2):
  xseq = _sc_seq(seq_input.reshape(-1), seq_embed_table)
  feats = _sc_feats(nodes, neigh_table.reshape(-1), embed_table)
  r = lambda v: v.reshape(1, -1)

  def bd(wf, wb):
    z = jnp.zeros((wf.shape[1] + wb.shape[1], 6 * H), jnp.float32)
    z = z.at[:wf.shape[1], :3 * H].set(wf.T).at[wf.shape[1]:, 3 * H:].set(wb.T)
    return z.astype(jnp.bfloat16)

  rep = lambda v, n: jnp.tile(v.reshape(-1, 1), (1, n))
  gweights = (A1_w, r(A1_b), r(A1_c), W_enc1,
              A2_w, r(A2_b), r(A2_c), W_enc2,
              Af_w, r(Af_b), rep(Af_c, E2), weight2)
  sweights = (
      gru_w_ih_l0f.T, gru_w_ih_l0b.T, r(gru_b_ih_l0f), r(gru_b_ih_l0b),
      bd(gru_w_hh_l0f, gru_w_hh_l0b),
      jnp.concatenate([gru_b_hh_l0f, gru_b_hh_l0b]).reshape(1, -1),
      gru_w_ih_l1f.T, gru_w_ih_l1b.T, r(gru_b_ih_l1f), r(gru_b_ih_l1b),
      bd(gru_w_hh_l1f, gru_w_hh_l1b),
      jnp.concatenate([gru_b_hh_l1f, gru_b_hh_l1b]).reshape(1, -1),
      weight1,
  )
  rnn = _seq_call(xseq, sweights)
  return _graph_call(feats, rnn, gweights)
